# baseline (speedup 1.0000x reference)
import functools

import jax
import jax.numpy as jnp
from jax import lax
from jax.experimental import pallas as pl
from jax.experimental.pallas import tpu as pltpu
from jax.experimental.pallas import tpu_sc as plsc

_NC = 2
_NS = 16
_NW = _NC * _NS
_LANES = 16
_BC = 128


def _project_body(emb_ref, w_ref, b_ref, out_ref):
    out_ref[...] = (
        lax.dot_general(
            emb_ref[...], w_ref[...], (((1,), (1,)), ((), ())),
            preferred_element_type=jnp.float32,
        )
        + b_ref[...]
    )


def _project_table(embedding, W, b):
    return pl.pallas_call(
        _project_body,
        out_shape=jax.ShapeDtypeStruct(embedding.shape, jnp.float32),
    )(embedding, W, b.reshape(1, -1))


def _make_gather(bsz, seq, d, table_words):
    cols_per_w = bsz // _NW
    n_chunks = cols_per_w // _BC
    assert bsz % (_NW * _BC) == 0
    n_lblocks = (seq + _LANES - 1) // _LANES
    last_l0 = seq - _LANES

    mesh = plsc.VectorSubcoreMesh(
        core_axis_name="c", subcore_axis_name="s",
        num_cores=_NC, num_subcores=_NS,
    )

    n_blocks = n_chunks * n_lblocks
    assert n_blocks % 2 == 0

    @functools.partial(
        pl.kernel,
        mesh=mesh,
        out_type=jax.ShapeDtypeStruct((seq, d, bsz), jnp.float32),
        scratch_types=[
            pltpu.VMEM((table_words,), jnp.float32),
            pltpu.VMEM((2, _LANES, _BC), jnp.int32),
            pltpu.VMEM((2, _LANES, d, _BC), jnp.float32),
            pltpu.SemaphoreType.DMA,
            pltpu.SemaphoreType.DMA,
            pltpu.SemaphoreType.DMA,
            pltpu.SemaphoreType.DMA,
        ],
        compiler_params=pltpu.CompilerParams(needs_layout_passes=False),
    )
    def gather_kernel(table_hbm, idxt_hbm, out_hbm, table_v, idx_v, blk_v,
                      isem0, isem1, osem0, osem1):
        wid = lax.axis_index("s") * _NC + lax.axis_index("c")
        wb = wid * cols_per_w
        pltpu.sync_copy(table_hbm, table_v)
        isems = (isem0, isem1)
        osems = (osem0, osem1)

        def block_coords(t):
            t = lax.min(t, n_blocks - 1)
            l0 = lax.min(lax.rem(t, n_lblocks) * _LANES, last_l0)
            bb = wb + lax.div(t, n_lblocks) * _BC
            return l0, bb

        def idx_copy(t, buf):
            l0, bb = block_coords(t)
            return pltpu.make_async_copy(
                idxt_hbm.at[pl.ds(l0, _LANES), pl.ds(bb, _BC)],
                idx_v.at[buf],
                isems[buf],
            )

        def out_copy(t, buf):
            l0, bb = block_coords(t)
            return pltpu.make_async_copy(
                blk_v.at[buf],
                out_hbm.at[pl.ds(l0, _LANES), :, pl.ds(bb, _BC)],
                osems[buf],
            )

        def compute(buf):
            @plsc.parallel_loop(0, _LANES, unroll=4)
            def lrow(l2):
                for bg in range(_BC // _LANES):
                    iv = idx_v[buf, l2, pl.ds(bg * _LANES, _LANES)]
                    a = iv * d
                    for o in range(d):
                        vals = plsc.load_gather(table_v, [a + o])
                        blk_v[buf, l2, o, pl.ds(bg * _LANES, _LANES)] = vals

        idx_copy(0, 0).start()
        idx_copy(1, 1).start()
        for b in (0, 1):
            idx_copy(b, b).wait()
            compute(b)
            out_copy(b, b).start()
            idx_copy(b + 2, b).start()

        def step(j, carry):
            for b in (0, 1):
                t = 2 * j + b
                idx_copy(t, b).wait()
                out_copy(t, b).wait()
                compute(b)
                out_copy(t, b).start()
                idx_copy(t + 2, b).start()
            return carry

        lax.fori_loop(1, n_blocks // 2, step, 0)

        for b in (0, 1):
            idx_copy(n_blocks - 2 + b, b).wait()
            out_copy(n_blocks - 2 + b, b).wait()

    return gather_kernel


def kernel(indices, embedding, W, b):
    bsz, seq = indices.shape
    d = embedding.shape[1]
    table = _project_table(embedding, W, b)
    flat = table.reshape(-1)
    table_words = (flat.shape[0] + 127) // 128 * 128
    flat = jnp.pad(flat, (0, table_words - flat.shape[0]))
    idx_t = jnp.transpose(indices.astype(jnp.int32))
    out_t = _make_gather(bsz, seq, d, table_words)(flat, idx_t)
    return jnp.transpose(out_t, (2, 0, 1))


# speedup vs baseline: 2.3588x; 2.3588x over previous
import functools

import jax
import jax.numpy as jnp
from jax import lax
from jax.experimental import pallas as pl
from jax.experimental.pallas import tpu as pltpu
from jax.experimental.pallas import tpu_sc as plsc

_NC = 2
_NS = 16
_NW = _NC * _NS
_LANES = 16
_BC = 128


def _project_body(emb_ref, w_ref, b_ref, out_ref):
    out_ref[...] = (
        lax.dot_general(
            emb_ref[...], w_ref[...], (((1,), (1,)), ((), ())),
            preferred_element_type=jnp.float32,
        )
        + b_ref[...]
    )


def _project_table(embedding, W, b):
    return pl.pallas_call(
        _project_body,
        out_shape=jax.ShapeDtypeStruct(embedding.shape, jnp.float32),
    )(embedding, W, b.reshape(1, -1))


def _make_gather(bsz, seq, d, table_words):
    cols_per_w = bsz // _NW
    n_chunks = cols_per_w // _BC
    assert bsz % (_NW * _BC) == 0
    n_lblocks = (seq + _LANES - 1) // _LANES
    last_l0 = seq - _LANES

    mesh = plsc.VectorSubcoreMesh(
        core_axis_name="c", subcore_axis_name="s",
        num_cores=_NC, num_subcores=_NS,
    )

    n_blocks = n_chunks * n_lblocks
    assert n_blocks % 2 == 0

    @functools.partial(
        pl.kernel,
        mesh=mesh,
        out_type=jax.ShapeDtypeStruct((seq, d, bsz), jnp.float32),
        scratch_types=[
            pltpu.VMEM((table_words,), jnp.float32),
            pltpu.VMEM((2, _LANES, _BC), jnp.int32),
            pltpu.VMEM((2, _LANES, d, _BC), jnp.float32),
            pltpu.SemaphoreType.DMA,
            pltpu.SemaphoreType.DMA,
            pltpu.SemaphoreType.DMA,
            pltpu.SemaphoreType.DMA,
        ],
        compiler_params=pltpu.CompilerParams(needs_layout_passes=False),
    )
    def gather_kernel(table_hbm, idxt_hbm, out_hbm, table_v, idx_v, blk_v,
                      isem0, isem1, osem0, osem1):
        wid = lax.axis_index("s") * _NC + lax.axis_index("c")
        wb = wid * cols_per_w
        pltpu.sync_copy(table_hbm, table_v)
        isems = (isem0, isem1)
        osems = (osem0, osem1)

        def block_coords(t):
            t = lax.min(t, n_blocks - 1)
            l0 = lax.min(lax.rem(t, n_lblocks) * _LANES, last_l0)
            bb = wb + lax.div(t, n_lblocks) * _BC
            return l0, bb

        def idx_copy(t, buf):
            l0, bb = block_coords(t)
            return pltpu.make_async_copy(
                idxt_hbm.at[pl.ds(l0, _LANES), pl.ds(bb, _BC)],
                idx_v.at[buf],
                isems[buf],
            )

        def out_copy(t, buf):
            l0, bb = block_coords(t)
            return pltpu.make_async_copy(
                blk_v.at[buf],
                out_hbm.at[pl.ds(l0, _LANES), :, pl.ds(bb, _BC)],
                osems[buf],
            )

        def compute(buf):
            @plsc.parallel_loop(0, _LANES * (_BC // _LANES), unroll=2)
            def lrow(it):
                l2 = lax.div(it, _BC // _LANES)
                bg = lax.rem(it, _BC // _LANES)
                iv = idx_v[buf, l2, pl.ds(bg * _LANES, _LANES)]
                a = iv * d
                for o in range(d):
                    vals = plsc.load_gather(table_v, [a + o])
                    blk_v[buf, l2, o, pl.ds(bg * _LANES, _LANES)] = vals

        idx_copy(0, 0).start()
        idx_copy(1, 1).start()
        for b in (0, 1):
            idx_copy(b, b).wait()
            compute(b)
            out_copy(b, b).start()
            idx_copy(b + 2, b).start()

        def step(j, carry):
            for b in (0, 1):
                t = 2 * j + b
                idx_copy(t, b).wait()
                out_copy(t, b).wait()
                compute(b)
                out_copy(t, b).start()
                idx_copy(t + 2, b).start()
            return carry

        lax.fori_loop(1, n_blocks // 2, step, 0)

        for b in (0, 1):
            idx_copy(n_blocks - 2 + b, b).wait()
            out_copy(n_blocks - 2 + b, b).wait()

    return gather_kernel


def kernel(indices, embedding, W, b):
    bsz, seq = indices.shape
    d = embedding.shape[1]
    table = _project_table(embedding, W, b)
    flat = table.reshape(-1)
    table_words = (flat.shape[0] + 127) // 128 * 128
    flat = jnp.pad(flat, (0, table_words - flat.shape[0]))
    idx_t = jnp.transpose(indices.astype(jnp.int32))
    out_t = _make_gather(bsz, seq, d, table_words)(flat, idx_t)
    return jnp.transpose(out_t, (2, 0, 1))
